# baseline (speedup 1.0000x reference)
import functools

import jax
import jax.numpy as jnp
from jax.experimental import pallas as pl
from jax.experimental.pallas import tpu as pltpu


def _gcn_kernel(x_ref, a_ref, w_ref, b_ref, o_ref, *, tile_t, v, bn):
    c_in = x_ref.shape[1]
    c_out = w_ref.shape[0]

    w = w_ref[...].astype(jnp.bfloat16)
    bcol = jnp.transpose(b_ref[...]).astype(jnp.bfloat16)
    w_aug = jnp.concatenate(
        [w, jnp.broadcast_to(bcol, (c_out, 8))], axis=1)
    aug_rows = jnp.full((8, tile_t * v), 0.125, dtype=jnp.bfloat16)

    for j in range(bn):
        a = a_ref[j].astype(jnp.bfloat16)
        xcat = x_ref[j].astype(jnp.bfloat16).reshape(c_in, tile_t * v)
        x_aug = jnp.concatenate([xcat, aug_rows], axis=0)
        y = jnp.dot(w_aug, x_aug,
                    preferred_element_type=jnp.float32).astype(jnp.bfloat16)
        ybig = y.reshape(c_out * tile_t, v)
        zrows = jnp.dot(ybig, a, preferred_element_type=jnp.float32)
        o_ref[j] = zrows.reshape(c_out, tile_t, v)


def _graph_conv(x, A, weight, bias, tile_t):
    n, c_in, t, v = x.shape
    c_out = weight.shape[0]

    while t % tile_t != 0:
        tile_t //= 2

    b2 = bias.reshape(1, c_out)
    bn = 2 if (n % 2 == 0 and tile_t == t) else 1

    body = functools.partial(_gcn_kernel, tile_t=tile_t, v=v, bn=bn)
    if tile_t == t:
        grid = (n // bn,)
        x_spec = pl.BlockSpec((bn, c_in, tile_t, v), lambda i: (i, 0, 0, 0))
        a_spec = pl.BlockSpec((bn, v, v), lambda i: (i, 0, 0))
        w_spec = pl.BlockSpec((c_out, c_in), lambda i: (0, 0))
        bias_spec = pl.BlockSpec((1, c_out), lambda i: (0, 0))
        o_spec = pl.BlockSpec((bn, c_out, tile_t, v), lambda i: (i, 0, 0, 0))
        semantics = ("parallel",)
    else:
        grid = (n // bn, t // tile_t)
        x_spec = pl.BlockSpec((bn, c_in, tile_t, v), lambda i, j: (i, 0, j, 0))
        a_spec = pl.BlockSpec((bn, v, v), lambda i, j: (i, 0, 0))
        w_spec = pl.BlockSpec((c_out, c_in), lambda i, j: (0, 0))
        bias_spec = pl.BlockSpec((1, c_out), lambda i, j: (0, 0))
        o_spec = pl.BlockSpec(
            (bn, c_out, tile_t, v), lambda i, j: (i, 0, j, 0))
        semantics = ("parallel", "parallel")
    out = pl.pallas_call(
        body,
        out_shape=jax.ShapeDtypeStruct((n, c_out, t, v), x.dtype),
        grid=grid,
        in_specs=[x_spec, a_spec, w_spec, bias_spec],
        out_specs=o_spec,
        compiler_params=pltpu.CompilerParams(
            dimension_semantics=semantics,
            vmem_limit_bytes=64 * 1024 * 1024,
        ),
    )(x, A, weight, b2)
    return out


def kernel(x, A, weight, bias):
    out = _graph_conv(x, A, weight, bias, tile_t=64)
    return out, A


# speedup vs baseline: 1.0052x; 1.0052x over previous
import functools

import jax
import jax.numpy as jnp
from jax.experimental import pallas as pl
from jax.experimental.pallas import tpu as pltpu


def _gcn_kernel(x_ref, a_ref, w_ref, b_ref, o_ref, *, tile_t, v, bn):
    c_in = x_ref.shape[1]
    c_out = w_ref.shape[0]

    w = w_ref[...].astype(jnp.bfloat16)
    bcol = jnp.transpose(b_ref[...]).astype(jnp.bfloat16)
    w_aug = jnp.concatenate(
        [w, jnp.broadcast_to(bcol, (c_out, 8))], axis=1)
    aug_rows = jnp.full((8, tile_t * v), 0.125, dtype=jnp.bfloat16)

    for j in range(bn):
        a = a_ref[j].astype(jnp.bfloat16)
        xcat = x_ref[j].astype(jnp.bfloat16).reshape(c_in, tile_t * v)
        x_aug = jnp.concatenate([xcat, aug_rows], axis=0)
        y = jnp.dot(w_aug, x_aug,
                    preferred_element_type=jnp.float32).astype(jnp.bfloat16)
        ybig = y.reshape(c_out * tile_t, v)
        zrows = jnp.dot(ybig, a, preferred_element_type=jnp.float32)
        o_ref[j] = zrows.reshape(c_out, tile_t, v)


def _graph_conv(x, A, weight, bias, tile_t):
    n, c_in, t, v = x.shape
    c_out = weight.shape[0]

    while t % tile_t != 0:
        tile_t //= 2

    b2 = bias.reshape(1, c_out)
    bn = 2 if (n % 2 == 0 and tile_t == t) else 1

    body = functools.partial(_gcn_kernel, tile_t=tile_t, v=v, bn=bn)
    if tile_t == t:
        grid = (n // bn,)
        x_spec = pl.BlockSpec((bn, c_in, tile_t, v), lambda i: (i, 0, 0, 0))
        a_spec = pl.BlockSpec((bn, v, v), lambda i: (i, 0, 0))
        w_spec = pl.BlockSpec((c_out, c_in), lambda i: (0, 0))
        bias_spec = pl.BlockSpec((1, c_out), lambda i: (0, 0))
        o_spec = pl.BlockSpec((bn, c_out, tile_t, v), lambda i: (i, 0, 0, 0))
        semantics = ("arbitrary",)
    else:
        grid = (n // bn, t // tile_t)
        x_spec = pl.BlockSpec((bn, c_in, tile_t, v), lambda i, j: (i, 0, j, 0))
        a_spec = pl.BlockSpec((bn, v, v), lambda i, j: (i, 0, 0))
        w_spec = pl.BlockSpec((c_out, c_in), lambda i, j: (0, 0))
        bias_spec = pl.BlockSpec((1, c_out), lambda i, j: (0, 0))
        o_spec = pl.BlockSpec(
            (bn, c_out, tile_t, v), lambda i, j: (i, 0, j, 0))
        semantics = ("parallel", "parallel")
    out = pl.pallas_call(
        body,
        out_shape=jax.ShapeDtypeStruct((n, c_out, t, v), x.dtype),
        grid=grid,
        in_specs=[x_spec, a_spec, w_spec, bias_spec],
        out_specs=o_spec,
        compiler_params=pltpu.CompilerParams(
            dimension_semantics=semantics,
            vmem_limit_bytes=64 * 1024 * 1024,
        ),
    )(x, A, weight, b2)
    return out


def kernel(x, A, weight, bias):
    out = _graph_conv(x, A, weight, bias, tile_t=64)
    return out, A
